# baseline (speedup 1.0000x reference)
import jax
import jax.numpy as jnp
from jax.experimental import pallas as pl
from jax.experimental.pallas import tpu as pltpu

_B, _N, _E = 16, 96, 9120
_NI, _NH, _NO, _NA = 128, 128, 128, 16
_T = 3040
_NB = _E // _T
_NP = 128
_BF = jnp.bfloat16


def _psi(p):
    return jnp.sign(p) * jnp.log(jnp.abs(p) + 1.0)


def _build_tables(h_ref, x_ref, xt_ref, wit_ref, wjt_ref, tabi_ref, tabj_ref,
                  xpad_ref):
    hb = h_ref[0]
    hpi = jnp.dot(hb, wit_ref[...], preferred_element_type=jnp.float32)
    hpj = jnp.dot(hb, wjt_ref[...], preferred_element_type=jnp.float32)

    xb = x_ref[0]
    xt = xt_ref[0]
    mrow = jnp.where(
        jax.lax.broadcasted_iota(jnp.int32, (1, 4), 1) == 0, 1.0, -1.0)
    mcol = jnp.where(
        jax.lax.broadcasted_iota(jnp.int32, (4, 1), 0) == 0, 1.0, -1.0)
    xg = xb * mrow
    dd = jnp.dot(xg, xt, preferred_element_type=jnp.float32)
    nv_c = jnp.sum(xg * xb, axis=1, keepdims=True)
    nv_r = jnp.sum(xt * xt * mcol, axis=0, keepdims=True)
    nsq = nv_c + nv_r - 2.0 * dd

    zc = jnp.zeros((_N, _NP - _N), _BF)
    zr = jnp.zeros((_NP - _N, 3 * _NP), _BF)
    tabi_ref[...] = jnp.concatenate([
        jnp.concatenate([hpi.astype(_BF), _psi(nsq).astype(_BF), zc,
                         _psi(dd).astype(_BF), zc], axis=1), zr], axis=0)
    tabj_ref[...] = jnp.concatenate(
        [hpj.astype(_BF), jnp.zeros((_NP - _N, _NH), _BF)], axis=0)
    xpad_ref[...] = jnp.concatenate([
        jnp.concatenate([xb.astype(_BF), jnp.zeros((_N, 4), _BF)], axis=1),
        jnp.zeros((_NP - _N, 8), _BF)], axis=0)


def _edge_core(nb, ei_ref, ej_ref, h_ref, x_ref, xt_ref, wit_ref, wjt_ref,
               wn128_ref, wd128_ref, tabi_ref, tabj_ref, xpad_ref):
    @pl.when(nb == 0)
    def _build():
        _build_tables(h_ref, x_ref, xt_ref, wit_ref, wjt_ref, tabi_ref,
                      tabj_ref, xpad_ref)

    ei = ei_ref[0, 0]
    ej = ej_ref[0, 0]
    iota_n = jax.lax.broadcasted_iota(jnp.int32, (_T, _NP), 1)
    ohi = (ei == iota_n).astype(_BF)
    ohj = (ej == iota_n).astype(_BF)
    gi = jnp.dot(ohi, tabi_ref[...], preferred_element_type=jnp.float32)
    gj = jnp.dot(ohj, tabj_ref[...], preferred_element_type=jnp.float32)
    s1 = (gi[:, _NP:2 * _NP] * ohj).astype(_BF)
    s2 = (gi[:, 2 * _NP:3 * _NP] * ohj).astype(_BF)
    z1 = (gi[:, 0:_NP] + gj
          + jnp.dot(s1, wn128_ref[...], preferred_element_type=jnp.float32)
          + jnp.dot(s2, wd128_ref[...], preferred_element_type=jnp.float32))
    return z1, ohi, ohj


def _pass1_body(ei_ref, ej_ref, h_ref, x_ref, xt_ref, wit_ref, wjt_ref,
                wn128_ref, wd128_ref, stats_ref, tabi_ref, tabj_ref,
                xpad_ref):
    b = pl.program_id(0)
    nb = pl.program_id(1)
    z1, _, _ = _edge_core(nb, ei_ref, ej_ref, h_ref, x_ref, xt_ref, wit_ref,
                          wjt_ref, wn128_ref, wd128_ref, tabi_ref, tabj_ref,
                          xpad_ref)

    @pl.when(jnp.logical_and(b == 0, nb == 0))
    def _init():
        stats_ref[...] = jnp.zeros((8, _NH), jnp.float32)

    stats_ref[0:1, :] = stats_ref[0:1, :] + jnp.sum(z1, axis=0, keepdims=True)
    stats_ref[1:2, :] = stats_ref[1:2, :] + jnp.sum(z1 * z1, axis=0,
                                                    keepdims=True)


def _pass2_body(ei_ref, ej_ref, eis_ref, h_ref, x_ref, xt_ref, wit_ref,
                wjt_ref, wn128_ref, wd128_ref, scale_ref, shift_ref, w2t_ref,
                b2_ref, wm_ref, bm_ref, wx1t_ref, bx1_ref, wx2_ref,
                m_ref, aggm_ref, aggx_ref, tabi_ref, tabj_ref, xpad_ref):
    nb = pl.program_id(1)
    z1, ohi, ohj = _edge_core(nb, ei_ref, ej_ref, h_ref, x_ref, xt_ref,
                              wit_ref, wjt_ref, wn128_ref, wd128_ref,
                              tabi_ref, tabj_ref, xpad_ref)
    z = jnp.maximum(z1 * scale_ref[...] + shift_ref[...], 0.0)
    mpre = jnp.maximum(
        jnp.dot(z, w2t_ref[...], preferred_element_type=jnp.float32)
        + b2_ref[...], 0.0)
    wgt = jax.nn.sigmoid(
        jnp.sum(mpre * wm_ref[...], axis=1, keepdims=True) + bm_ref[...])
    m = mpre * wgt
    m_ref[0] = m
    y = jnp.maximum(
        jnp.dot(m, wx1t_ref[...], preferred_element_type=jnp.float32)
        + bx1_ref[...], 0.0)
    px = jnp.sum(y * wx2_ref[...], axis=1, keepdims=True)
    xdp = jnp.dot(ohi - ohj, xpad_ref[...],
                  preferred_element_type=jnp.float32)
    trans = jnp.clip(xdp * px, -100.0, 100.0)
    onecol = jnp.where(
        jax.lax.broadcasted_iota(jnp.int32, (1, 8), 1) == 4, 1.0, 0.0)
    tp8 = (trans + onecol).astype(_BF)

    eis = eis_ref[0, 0]
    iota_t = jax.lax.broadcasted_iota(jnp.int32, (_N, _T), 0)
    ohit = (eis == iota_t).astype(_BF)
    am = jnp.dot(ohit, m.astype(_BF),
                 preferred_element_type=jnp.float32)
    ax = jnp.dot(ohit, tp8, preferred_element_type=jnp.float32)

    @pl.when(nb == 0)
    def _init():
        aggm_ref[0] = am
        aggx_ref[0] = ax

    @pl.when(nb != 0)
    def _acc():
        aggm_ref[0] = aggm_ref[0] + am
        aggx_ref[0] = aggx_ref[0] + ax


def _node_body(h_ref, aggm_ref, na_ref, x_ref, aggx_ref, wh1h_ref, wh1m_ref,
               wh1a_ref, bh1_ref, gh_ref, bh_ref, wh2t_ref, bh2_ref,
               hout_ref, xout_ref):
    h2 = h_ref[...].reshape(_B * _N, _NI)
    am2 = aggm_ref[...].reshape(_B * _N, _NH)
    na2 = na_ref[...].reshape(_B * _N, _NA)
    z = (jnp.dot(h2, wh1h_ref[...], preferred_element_type=jnp.float32)
         + jnp.dot(am2, wh1m_ref[...], preferred_element_type=jnp.float32)
         + jnp.dot(na2, wh1a_ref[...], preferred_element_type=jnp.float32)
         + bh1_ref[...])
    mu = jnp.mean(z, axis=0, keepdims=True)
    zc = z - mu
    var = jnp.mean(zc * zc, axis=0, keepdims=True)
    zn = gh_ref[...] * zc * jax.lax.rsqrt(var + 1e-5) + bh_ref[...]
    zr = jnp.maximum(zn, 0.0)
    z2 = (jnp.dot(zr, wh2t_ref[...], preferred_element_type=jnp.float32)
          + bh2_ref[...])
    hout_ref[...] = h_ref[...] + z2.reshape(_B, _N, _NO)
    cnt = aggx_ref[:, :, 4:5]
    xout_ref[...] = x_ref[...] + aggx_ref[:, :, 0:4] / jnp.maximum(cnt, 1.0)


def kernel(h, x, edgei, edgej, node_attr, W1, g1, b1, W2, b2, Wh1, bh1, gh,
           bh, Wh2, bh2, Wx1, bx1, Wx2, Wm, bm):
    f32 = jnp.float32
    wit = W1[:, :_NI].T
    wjt = W1[:, _NI:2 * _NI].T
    wn128 = jnp.broadcast_to(W1[:, 2 * _NI][None, :], (_NP, _NH)).astype(_BF)
    wd128 = jnp.broadcast_to(W1[:, 2 * _NI + 1][None, :],
                             (_NP, _NH)).astype(_BF)
    xt = x.transpose(0, 2, 1)
    ei_g = edgei.reshape(_B, _NB, _T, 1)
    ej_g = edgej.reshape(_B, _NB, _T, 1)
    ei_s = edgei.reshape(_B, _NB, 1, _T)

    edge_fixed_specs = [
        pl.BlockSpec((1, 1, _T, 1), lambda b, nb: (b, nb, 0, 0)),
        pl.BlockSpec((1, 1, _T, 1), lambda b, nb: (b, nb, 0, 0)),
        pl.BlockSpec((1, _N, _NI), lambda b, nb: (b, 0, 0)),
        pl.BlockSpec((1, _N, 4), lambda b, nb: (b, 0, 0)),
        pl.BlockSpec((1, 4, _N), lambda b, nb: (b, 0, 0)),
        pl.BlockSpec((_NI, _NH), lambda b, nb: (0, 0)),
        pl.BlockSpec((_NI, _NH), lambda b, nb: (0, 0)),
        pl.BlockSpec((_NP, _NH), lambda b, nb: (0, 0)),
        pl.BlockSpec((_NP, _NH), lambda b, nb: (0, 0)),
    ]
    row_spec = pl.BlockSpec((1, _NH), lambda b, nb: (0, 0))
    scratch = [pltpu.VMEM((_NP, 3 * _NP), _BF),
               pltpu.VMEM((_NP, _NH), _BF),
               pltpu.VMEM((_NP, 8), _BF)]

    stats = pl.pallas_call(
        _pass1_body,
        grid=(_B, _NB),
        in_specs=edge_fixed_specs,
        out_specs=pl.BlockSpec((8, _NH), lambda b, nb: (0, 0)),
        out_shape=jax.ShapeDtypeStruct((8, _NH), f32),
        scratch_shapes=scratch,
    )(ei_g, ej_g, h, x, xt, wit, wjt, wn128, wd128)

    r = float(_B * _E)
    mu = stats[0] / r
    var = stats[1] / r - mu * mu
    scale_v = g1 * jax.lax.rsqrt(var + 1e-5)
    shift_v = b1 - mu * scale_v

    m, aggm, aggx = pl.pallas_call(
        _pass2_body,
        grid=(_B, _NB),
        in_specs=(edge_fixed_specs[:2]
                  + [pl.BlockSpec((1, 1, 1, _T), lambda b, nb: (b, nb, 0, 0))]
                  + edge_fixed_specs[2:]
                  + [row_spec, row_spec,
                     pl.BlockSpec((_NH, _NH), lambda b, nb: (0, 0)),
                     row_spec,
                     row_spec,
                     pl.BlockSpec((1, 1), lambda b, nb: (0, 0)),
                     pl.BlockSpec((_NH, _NH), lambda b, nb: (0, 0)),
                     row_spec,
                     row_spec]),
        out_specs=[
            pl.BlockSpec((1, _T, _NH), lambda b, nb: (b, nb, 0)),
            pl.BlockSpec((1, _N, _NH), lambda b, nb: (b, 0, 0)),
            pl.BlockSpec((1, _N, 8), lambda b, nb: (b, 0, 0)),
        ],
        out_shape=[
            jax.ShapeDtypeStruct((_B, _E, _NH), f32),
            jax.ShapeDtypeStruct((_B, _N, _NH), f32),
            jax.ShapeDtypeStruct((_B, _N, 8), f32),
        ],
        scratch_shapes=scratch,
    )(ei_g, ej_g, ei_s, h, x, xt, wit, wjt, wn128, wd128,
      scale_v[None, :], shift_v[None, :], W2.T, b2[None, :], Wm,
      bm.reshape(1, 1), Wx1.T, bx1[None, :], Wx2)

    h_out, x_out = pl.pallas_call(
        _node_body,
        out_shape=[
            jax.ShapeDtypeStruct((_B, _N, _NO), f32),
            jax.ShapeDtypeStruct((_B, _N, 4), f32),
        ],
    )(h, aggm, node_attr, x, aggx, Wh1[:, :_NI].T, Wh1[:, _NI:_NI + _NH].T,
      Wh1[:, _NI + _NH:].T, bh1[None, :], gh[None, :], bh[None, :], Wh2.T,
      bh2[None, :])

    return (h_out, x_out, m)


# speedup vs baseline: 1.3314x; 1.3314x over previous
import jax
import jax.numpy as jnp
from jax.experimental import pallas as pl
from jax.experimental.pallas import tpu as pltpu

_B, _N, _E = 16, 96, 9120
_NI, _NH, _NO, _NA = 128, 128, 128, 16
_T = 3040
_NB = _E // _T
_NP = 128
_BF = jnp.bfloat16


def _psi(p):
    return jnp.sign(p) * jnp.log(jnp.abs(p) + 1.0)


def _build_tables(h_ref, x_ref, xt_ref, wit_ref, wjt_ref, tabi_ref, tabj_ref,
                  xpad_ref):
    hb = h_ref[0]
    hpi = jnp.dot(hb, wit_ref[...], preferred_element_type=jnp.float32)
    hpj = jnp.dot(hb, wjt_ref[...], preferred_element_type=jnp.float32)

    xb = x_ref[0]
    xt = xt_ref[0]
    mrow = jnp.where(
        jax.lax.broadcasted_iota(jnp.int32, (1, 4), 1) == 0, 1.0, -1.0)
    mcol = jnp.where(
        jax.lax.broadcasted_iota(jnp.int32, (4, 1), 0) == 0, 1.0, -1.0)
    xg = xb * mrow
    dd = jnp.dot(xg, xt, preferred_element_type=jnp.float32)
    nv_c = jnp.sum(xg * xb, axis=1, keepdims=True)
    nv_r = jnp.sum(xt * xt * mcol, axis=0, keepdims=True)
    nsq = nv_c + nv_r - 2.0 * dd

    zc = jnp.zeros((_N, _NP - _N), _BF)
    zr = jnp.zeros((_NP - _N, 3 * _NP), _BF)
    tabi_ref[...] = jnp.concatenate([
        jnp.concatenate([hpi.astype(_BF), _psi(nsq).astype(_BF), zc,
                         _psi(dd).astype(_BF), zc], axis=1), zr], axis=0)
    tabj_ref[...] = jnp.concatenate(
        [hpj.astype(_BF), jnp.zeros((_NP - _N, _NH), _BF)], axis=0)
    xpad_ref[...] = jnp.concatenate([
        jnp.concatenate([xb.astype(_BF), jnp.zeros((_N, 4), _BF)], axis=1),
        jnp.zeros((_NP - _N, 8), _BF)], axis=0)


def _edge_core(nb, ei_ref, ej_ref, h_ref, x_ref, xt_ref, wit_ref, wjt_ref,
               wn128_ref, wd128_ref, tabi_ref, tabj_ref, xpad_ref):
    @pl.when(nb == 0)
    def _build():
        _build_tables(h_ref, x_ref, xt_ref, wit_ref, wjt_ref, tabi_ref,
                      tabj_ref, xpad_ref)

    ei = ei_ref[0, 0]
    ej = ej_ref[0, 0]
    iota_n = jax.lax.broadcasted_iota(jnp.int32, (_T, _NP), 1)
    ohi = (ei == iota_n).astype(_BF)
    ohj = (ej == iota_n).astype(_BF)
    gi = jnp.dot(ohi, tabi_ref[...], preferred_element_type=jnp.float32)
    gj = jnp.dot(ohj, tabj_ref[...], preferred_element_type=jnp.float32)
    s1 = (gi[:, _NP:2 * _NP] * ohj).astype(_BF)
    s2 = (gi[:, 2 * _NP:3 * _NP] * ohj).astype(_BF)
    z1 = (gi[:, 0:_NP] + gj
          + jnp.dot(s1, wn128_ref[...], preferred_element_type=jnp.float32)
          + jnp.dot(s2, wd128_ref[...], preferred_element_type=jnp.float32))
    return z1, ohi, ohj


def _pass1_body(ei_ref, ej_ref, h_ref, x_ref, xt_ref, wit_ref, wjt_ref,
                wn128_ref, wd128_ref, stats_ref, tabi_ref, tabj_ref,
                xpad_ref):
    b = pl.program_id(0)
    nb = pl.program_id(1)
    z1, _, _ = _edge_core(nb, ei_ref, ej_ref, h_ref, x_ref, xt_ref, wit_ref,
                          wjt_ref, wn128_ref, wd128_ref, tabi_ref, tabj_ref,
                          xpad_ref)

    @pl.when(jnp.logical_and(b == 0, nb == 0))
    def _init():
        stats_ref[...] = jnp.zeros((8, _NH), jnp.float32)

    stats_ref[0:1, :] = stats_ref[0:1, :] + jnp.sum(z1, axis=0, keepdims=True)
    stats_ref[1:2, :] = stats_ref[1:2, :] + jnp.sum(z1 * z1, axis=0,
                                                    keepdims=True)


def _pass2_body(ei_ref, ej_ref, eis_ref, h_ref, x_ref, xt_ref, wit_ref,
                wjt_ref, wn128_ref, wd128_ref, scale_ref, shift_ref, w2t_ref,
                b2_ref, wm_ref, bm_ref, wx1t_ref, bx1_ref, wx2_ref,
                m_ref, aggm_ref, aggx_ref, tabi_ref, tabj_ref, xpad_ref):
    nb = pl.program_id(1)
    z1, ohi, ohj = _edge_core(nb, ei_ref, ej_ref, h_ref, x_ref, xt_ref,
                              wit_ref, wjt_ref, wn128_ref, wd128_ref,
                              tabi_ref, tabj_ref, xpad_ref)
    z = jnp.maximum(z1 * scale_ref[...] + shift_ref[...], 0.0)
    mpre = jnp.maximum(
        jnp.dot(z.astype(_BF), w2t_ref[...],
                preferred_element_type=jnp.float32)
        + b2_ref[...], 0.0)
    mpre_bf = mpre.astype(_BF)
    logit = jnp.dot(mpre_bf, wm_ref[...],
                    preferred_element_type=jnp.float32) + bm_ref[...]
    m = mpre * jax.nn.sigmoid(logit)
    m_ref[0] = m
    y = jnp.maximum(
        jnp.dot(m.astype(_BF), wx1t_ref[...],
                preferred_element_type=jnp.float32)
        + bx1_ref[...], 0.0)
    pxf = jnp.dot(y.astype(_BF), wx2_ref[...],
                  preferred_element_type=jnp.float32)
    xdp = jnp.dot(ohi - ohj, xpad_ref[...],
                  preferred_element_type=jnp.float32)
    trans = jnp.clip(xdp * pxf, -100.0, 100.0)
    onecol = jnp.where(
        jax.lax.broadcasted_iota(jnp.int32, (1, 8), 1) == 4, 1.0, 0.0)
    tp8 = (trans + onecol).astype(_BF)

    eis = eis_ref[0, 0]
    iota_t = jax.lax.broadcasted_iota(jnp.int32, (_N, _T), 0)
    ohit = (eis == iota_t).astype(_BF)
    am = jnp.dot(ohit, m.astype(_BF),
                 preferred_element_type=jnp.float32)
    ax = jnp.dot(ohit, tp8, preferred_element_type=jnp.float32)

    @pl.when(nb == 0)
    def _init():
        aggm_ref[0] = am
        aggx_ref[0] = ax

    @pl.when(nb != 0)
    def _acc():
        aggm_ref[0] = aggm_ref[0] + am
        aggx_ref[0] = aggx_ref[0] + ax


def _node_body(h_ref, aggm_ref, na_ref, x_ref, aggx_ref, wh1h_ref, wh1m_ref,
               wh1a_ref, bh1_ref, gh_ref, bh_ref, wh2t_ref, bh2_ref,
               hout_ref, xout_ref):
    h2 = h_ref[...].reshape(_B * _N, _NI)
    am2 = aggm_ref[...].reshape(_B * _N, _NH)
    na2 = na_ref[...].reshape(_B * _N, _NA)
    z = (jnp.dot(h2, wh1h_ref[...], preferred_element_type=jnp.float32)
         + jnp.dot(am2, wh1m_ref[...], preferred_element_type=jnp.float32)
         + jnp.dot(na2, wh1a_ref[...], preferred_element_type=jnp.float32)
         + bh1_ref[...])
    mu = jnp.mean(z, axis=0, keepdims=True)
    zc = z - mu
    var = jnp.mean(zc * zc, axis=0, keepdims=True)
    zn = gh_ref[...] * zc * jax.lax.rsqrt(var + 1e-5) + bh_ref[...]
    zr = jnp.maximum(zn, 0.0)
    z2 = (jnp.dot(zr, wh2t_ref[...], preferred_element_type=jnp.float32)
          + bh2_ref[...])
    hout_ref[...] = h_ref[...] + z2.reshape(_B, _N, _NO)
    cnt = aggx_ref[:, :, 4:5]
    xout_ref[...] = x_ref[...] + aggx_ref[:, :, 0:4] / jnp.maximum(cnt, 1.0)


def kernel(h, x, edgei, edgej, node_attr, W1, g1, b1, W2, b2, Wh1, bh1, gh,
           bh, Wh2, bh2, Wx1, bx1, Wx2, Wm, bm):
    f32 = jnp.float32
    wit = W1[:, :_NI].T
    wjt = W1[:, _NI:2 * _NI].T
    wn128 = jnp.broadcast_to(W1[:, 2 * _NI][None, :], (_NP, _NH)).astype(_BF)
    wd128 = jnp.broadcast_to(W1[:, 2 * _NI + 1][None, :],
                             (_NP, _NH)).astype(_BF)
    xt = x.transpose(0, 2, 1)
    ei_g = edgei.reshape(_B, _NB, _T, 1)
    ej_g = edgej.reshape(_B, _NB, _T, 1)
    ei_s = edgei.reshape(_B, _NB, 1, _T)

    edge_fixed_specs = [
        pl.BlockSpec((1, 1, _T, 1), lambda b, nb: (b, nb, 0, 0)),
        pl.BlockSpec((1, 1, _T, 1), lambda b, nb: (b, nb, 0, 0)),
        pl.BlockSpec((1, _N, _NI), lambda b, nb: (b, 0, 0)),
        pl.BlockSpec((1, _N, 4), lambda b, nb: (b, 0, 0)),
        pl.BlockSpec((1, 4, _N), lambda b, nb: (b, 0, 0)),
        pl.BlockSpec((_NI, _NH), lambda b, nb: (0, 0)),
        pl.BlockSpec((_NI, _NH), lambda b, nb: (0, 0)),
        pl.BlockSpec((_NP, _NH), lambda b, nb: (0, 0)),
        pl.BlockSpec((_NP, _NH), lambda b, nb: (0, 0)),
    ]
    row_spec = pl.BlockSpec((1, _NH), lambda b, nb: (0, 0))
    scratch = [pltpu.VMEM((_NP, 3 * _NP), _BF),
               pltpu.VMEM((_NP, _NH), _BF),
               pltpu.VMEM((_NP, 8), _BF)]

    stats = pl.pallas_call(
        _pass1_body,
        grid=(_B, _NB),
        in_specs=edge_fixed_specs,
        out_specs=pl.BlockSpec((8, _NH), lambda b, nb: (0, 0)),
        out_shape=jax.ShapeDtypeStruct((8, _NH), f32),
        scratch_shapes=scratch,
    )(ei_g, ej_g, h, x, xt, wit, wjt, wn128, wd128)

    r = float(_B * _E)
    mu = stats[0] / r
    var = stats[1] / r - mu * mu
    scale_v = g1 * jax.lax.rsqrt(var + 1e-5)
    shift_v = b1 - mu * scale_v

    m, aggm, aggx = pl.pallas_call(
        _pass2_body,
        grid=(_B, _NB),
        in_specs=(edge_fixed_specs[:2]
                  + [pl.BlockSpec((1, 1, 1, _T), lambda b, nb: (b, nb, 0, 0))]
                  + edge_fixed_specs[2:]
                  + [row_spec, row_spec,
                     pl.BlockSpec((_NH, _NH), lambda b, nb: (0, 0)),
                     row_spec,
                     pl.BlockSpec((_NH, _NH), lambda b, nb: (0, 0)),
                     pl.BlockSpec((1, 1), lambda b, nb: (0, 0)),
                     pl.BlockSpec((_NH, _NH), lambda b, nb: (0, 0)),
                     row_spec,
                     pl.BlockSpec((_NH, 8), lambda b, nb: (0, 0))]),
        out_specs=[
            pl.BlockSpec((1, _T, _NH), lambda b, nb: (b, nb, 0)),
            pl.BlockSpec((1, _N, _NH), lambda b, nb: (b, 0, 0)),
            pl.BlockSpec((1, _N, 8), lambda b, nb: (b, 0, 0)),
        ],
        out_shape=[
            jax.ShapeDtypeStruct((_B, _E, _NH), f32),
            jax.ShapeDtypeStruct((_B, _N, _NH), f32),
            jax.ShapeDtypeStruct((_B, _N, 8), f32),
        ],
        scratch_shapes=scratch,
    )(ei_g, ej_g, ei_s, h, x, xt, wit, wjt, wn128, wd128,
      scale_v[None, :], shift_v[None, :], W2.T.astype(_BF), b2[None, :],
      jnp.broadcast_to(Wm.T, (_NH, _NH)).astype(_BF), bm.reshape(1, 1),
      Wx1.T.astype(_BF), bx1[None, :],
      jnp.broadcast_to(Wx2.T, (_NH, 8)).astype(_BF))

    h_out, x_out = pl.pallas_call(
        _node_body,
        out_shape=[
            jax.ShapeDtypeStruct((_B, _N, _NO), f32),
            jax.ShapeDtypeStruct((_B, _N, 4), f32),
        ],
    )(h, aggm, node_attr, x, aggx, Wh1[:, :_NI].T, Wh1[:, _NI:_NI + _NH].T,
      Wh1[:, _NI + _NH:].T, bh1[None, :], gh[None, :], bh[None, :], Wh2.T,
      bh2[None, :])

    return (h_out, x_out, m)
